# baseline (speedup 1.0000x reference)
import jax
import jax.numpy as jnp
from jax import lax
from jax.experimental import pallas as pl
from jax.experimental.pallas import tpu as pltpu
from jax.experimental.pallas import tpu_sc as plsc

B, C, H, W = 4, 96, 384, 384
HW = H * W
V = B * HW
L = 16
NC, NS = 2, 16
NW = NC * NS
RPW = H // (NW // B)
CHUNK = 128
SUBS = W // CHUNK
NCHUNK = RPW * SUBS
NG = CHUNK // L
CP = C // 2
CW = CP


def _inc(y, s):
    last = s == SUBS - 1
    return jnp.where(last, y + 1, y), jnp.where(last, 0, s + 1)


def _warp_body(table, fx, fy, out_hbm,
               fxv, fyv, alv, bev, idx, oidx, rows, outv,
               gsem, fsem, osem):
    wid = lax.axis_index("s") * NC + lax.axis_index("c")
    b = lax.shift_right_logical(wid, 3)
    r0 = (wid & 7) * RPW
    bhw = b * HW
    iota = lax.iota(jnp.int32, L)

    def flow_fire(y, s, p):
        off = bhw + y * W + s * CHUNK
        pltpu.async_copy(fx.at[pl.ds(off, CHUNK)], fxv[p], fsem[p])
        pltpu.async_copy(fy.at[pl.ds(off, CHUNK)], fyv[p], fsem[p])

    def flow_wait(p):
        pltpu.make_async_copy(fx.at[pl.ds(0, CHUNK)], fxv[p], fsem[p]).wait()
        pltpu.make_async_copy(fy.at[pl.ds(0, CHUNK)], fyv[p], fsem[p]).wait()

    def idx_and_fire(y, s, p):
        flow_wait(p)
        xoff = s * CHUNK
        yv = jnp.full((L,), y, jnp.int32)
        for k in range(NG):
            sl = pl.ds(k * L, L)
            xi = xoff + (k * L) + iota
            xf = xi.astype(jnp.float32) + fxv[p][sl]
            yf = yv.astype(jnp.float32) + fyv[p][sl]
            ix0 = xf.astype(jnp.int32)
            ix0 = jnp.where(ix0.astype(jnp.float32) > xf, ix0 - 1, ix0)
            iy0 = yf.astype(jnp.int32)
            iy0 = jnp.where(iy0.astype(jnp.float32) > yf, iy0 - 1, iy0)
            ixL = jnp.clip(ix0, 0, W - 1)
            iyT = jnp.clip(iy0, 0, H - 1)
            ixR = jnp.minimum(ixL + 1, W - 1)
            iyB = jnp.minimum(iyT + 1, H - 1)
            alv[p][sl] = xf - ixL.astype(jnp.float32)
            bev[p][sl] = yf - iyT.astype(jnp.float32)
            rowT = bhw + iyT * W
            rowB = bhw + iyB * W
            idx[p][0][sl] = rowT + ixL
            idx[p][1][sl] = rowT + ixR
            idx[p][2][sl] = rowB + ixL
            idx[p][3][sl] = rowB + ixR
        for q in range(4):
            pltpu.async_copy(table.at[idx[p][q]], rows[p][q], gsem[p])

    def gather_wait(p):
        for q in range(4):
            pltpu.make_async_copy(table.at[idx[p][q]], rows[p][q],
                                  gsem[p]).wait()

    def out_wait(p):
        pltpu.make_async_copy(outv[p], out_hbm.at[oidx[p]], osem[p]).wait()

    def blend_and_out(y, s, p, t):
        gather_wait(p)

        @pl.when(t > 0)
        def _():
            out_wait(p)

        yv = jnp.full((L,), y, jnp.int32)
        for g in range(C // L):
            cvec = g * L + iota
            oidx[p][pl.ds(g * L, L)] = ((b * C + cvec) * H + yv) * SUBS + s

        rtl, rtr, rbl, rbr = rows[p]
        ov = outv[p]
        fmt = plsc.PackFormat.INTERLEAVED
        for k in range(NG):
            sl = pl.ds(k * L, L)
            r = k * L + iota
            al = alv[p][sl]
            be = bev[p][sl]

            @plsc.parallel_loop(0, CP, unroll=4)
            def _blend(w, r=r, al=al, be=be, sl=sl):
                wspl = jnp.full((L,), w, jnp.int32)
                tl = plsc.bitcast(plsc.load_gather(rtl, [r, wspl]),
                                  jnp.bfloat16)
                tr = plsc.bitcast(plsc.load_gather(rtr, [r, wspl]),
                                  jnp.bfloat16)
                bl = plsc.bitcast(plsc.load_gather(rbl, [r, wspl]),
                                  jnp.bfloat16)
                br = plsc.bitcast(plsc.load_gather(rbr, [r, wspl]),
                                  jnp.bfloat16)
                tl_e, tl_o = plsc.unpack(tl, format=fmt)
                tr_e, tr_o = plsc.unpack(tr, format=fmt)
                bl_e, bl_o = plsc.unpack(bl, format=fmt)
                br_e, br_o = plsc.unpack(br, format=fmt)
                top_e = tl_e + al * (tr_e - tl_e)
                bot_e = bl_e + al * (br_e - bl_e)
                ov[2 * w, sl] = top_e + be * (bot_e - top_e)
                top_o = tl_o + al * (tr_o - tl_o)
                bot_o = bl_o + al * (br_o - bl_o)
                ov[2 * w + 1, sl] = top_o + be * (bot_o - top_o)

        pltpu.async_copy(ov, out_hbm.at[oidx[p]], osem[p])

    y0 = r0 + jnp.int32(0)
    s0 = jnp.int32(0)
    flow_fire(y0, s0, 0)
    y1, s1 = _inc(y0, s0)
    flow_fire(y1, s1, 1)
    idx_and_fire(y0, s0, 0)

    def body(t, carry):
        ya, sa = carry
        yb, sb = _inc(ya, sa)
        yc, sc = _inc(yb, sb)
        yd, sd = _inc(yc, sc)
        last = t >= NCHUNK // 2 - 1
        ycc = jnp.where(last, ya, yc)
        scc = jnp.where(last, sa, sc)
        ydc = jnp.where(last, yb, yd)
        sdc = jnp.where(last, sb, sd)
        flow_fire(ycc, scc, 0)
        idx_and_fire(yb, sb, 1)
        blend_and_out(ya, sa, 0, t)
        flow_fire(ydc, sdc, 1)
        idx_and_fire(ycc, scc, 0)
        blend_and_out(yb, sb, 1, t)
        return yc, sc

    lax.fori_loop(0, NCHUNK // 2, body, (y0, s0))
    flow_wait(1)
    gather_wait(0)
    out_wait(0)
    out_wait(1)


_warp = pl.kernel(
    _warp_body,
    out_type=jax.ShapeDtypeStruct((B * C * H * SUBS, CHUNK), jnp.float32),
    compiler_params=pltpu.CompilerParams(
        needs_layout_passes=False, use_tc_tiling_on_sc=False),
    mesh=plsc.VectorSubcoreMesh(core_axis_name="c", subcore_axis_name="s"),
    scratch_types=[
        [pltpu.VMEM((CHUNK,), jnp.float32) for _ in range(2)],
        [pltpu.VMEM((CHUNK,), jnp.float32) for _ in range(2)],
        [pltpu.VMEM((CHUNK,), jnp.float32) for _ in range(2)],
        [pltpu.VMEM((CHUNK,), jnp.float32) for _ in range(2)],
        [[pltpu.VMEM((CHUNK,), jnp.int32) for _ in range(4)]
         for _ in range(2)],
        [pltpu.VMEM((C,), jnp.int32) for _ in range(2)],
        [[pltpu.VMEM((CHUNK, CW), jnp.int32) for _ in range(4)]
         for _ in range(2)],
        [pltpu.VMEM((C, CHUNK), jnp.float32) for _ in range(2)],
        [pltpu.SemaphoreType.DMA for _ in range(2)],
        [pltpu.SemaphoreType.DMA for _ in range(2)],
        [pltpu.SemaphoreType.DMA for _ in range(2)],
    ],
)


def kernel(input1, input2):
    t = input1.transpose(0, 2, 3, 1).astype(jnp.bfloat16)
    table = lax.bitcast_convert_type(t.reshape(V, CW, 2), jnp.int32)
    fx = input2[:, 0, :, :].reshape(V)
    fy = input2[:, 1, :, :].reshape(V)
    out = _warp(table, fx, fy)
    return out.reshape(B, C, H, W)


# speedup vs baseline: 1.2613x; 1.2613x over previous
import jax
import jax.numpy as jnp
from jax import lax
from jax.experimental import pallas as pl
from jax.experimental.pallas import tpu as pltpu
from jax.experimental.pallas import tpu_sc as plsc

B, C, H, W = 4, 96, 384, 384
HW = H * W
V = B * HW
L = 16
NC, NS = 2, 16
NW = NC * NS
RPW = H // (NW // B)
CHUNK = 128
SUBS = W // CHUNK
NCHUNK = RPW * SUBS
NG = CHUNK // L
CP = C // 2
CW = CP


def _inc(y, s):
    last = s == SUBS - 1
    return jnp.where(last, y + 1, y), jnp.where(last, 0, s + 1)


def _warp_body(table, fx, fy, out_hbm,
               fxv, fyv, alv, bev, idx, oidx, rows, outv,
               gsem, fsem, osem):
    wid = lax.axis_index("s") * NC + lax.axis_index("c")
    b = lax.shift_right_logical(wid, 3)
    r0 = (wid & 7) * RPW
    bhw = b * HW
    iota = lax.iota(jnp.int32, L)

    def flow_fire(y, s, p):
        off = bhw + y * W + s * CHUNK
        pltpu.async_copy(fx.at[pl.ds(off, CHUNK)], fxv[p], fsem[p])
        pltpu.async_copy(fy.at[pl.ds(off, CHUNK)], fyv[p], fsem[p])

    def flow_wait(p):
        pltpu.make_async_copy(fx.at[pl.ds(0, CHUNK)], fxv[p], fsem[p]).wait()
        pltpu.make_async_copy(fy.at[pl.ds(0, CHUNK)], fyv[p], fsem[p]).wait()

    def idx_and_fire(y, s, p):
        flow_wait(p)
        xoff = s * CHUNK
        yv = jnp.full((L,), y, jnp.int32)
        for k in range(NG):
            sl = pl.ds(k * L, L)
            xi = xoff + (k * L) + iota
            xf = xi.astype(jnp.float32) + fxv[p][sl]
            yf = yv.astype(jnp.float32) + fyv[p][sl]
            ix0 = xf.astype(jnp.int32)
            ix0 = jnp.where(ix0.astype(jnp.float32) > xf, ix0 - 1, ix0)
            iy0 = yf.astype(jnp.int32)
            iy0 = jnp.where(iy0.astype(jnp.float32) > yf, iy0 - 1, iy0)
            ixL = jnp.clip(ix0, 0, W - 1)
            iyT = jnp.clip(iy0, 0, H - 1)
            ixR = jnp.minimum(ixL + 1, W - 1)
            iyB = jnp.minimum(iyT + 1, H - 1)
            alv[p][sl] = xf - ixL.astype(jnp.float32)
            bev[p][sl] = yf - iyT.astype(jnp.float32)
            rowT = bhw + iyT * W
            rowB = bhw + iyB * W
            idx[p][0][sl] = rowT + ixL
            idx[p][1][sl] = rowT + ixR
            idx[p][2][sl] = rowB + ixL
            idx[p][3][sl] = rowB + ixR
        for q in range(4):
            pltpu.async_copy(table.at[idx[p][q]], rows[p][q], gsem[p])

    def gather_wait(p):
        for q in range(4):
            pltpu.make_async_copy(table.at[idx[p][q]], rows[p][q],
                                  gsem[p]).wait()

    def out_wait(p):
        pltpu.make_async_copy(outv[p], out_hbm.at[oidx[p]], osem[p]).wait()

    def blend_and_out(y, s, p, t):
        gather_wait(p)

        @pl.when(t > 0)
        def _():
            out_wait(p)

        yv = jnp.full((L,), y, jnp.int32)
        for g in range(C // L):
            cvec = g * L + iota
            oidx[p][pl.ds(g * L, L)] = ((b * C + cvec) * H + yv) * SUBS + s

        rtl, rtr, rbl, rbr = rows[p]
        ov = outv[p]
        fmt = plsc.PackFormat.INTERLEAVED
        for k in range(NG):
            sl = pl.ds(k * L, L)
            r = k * L + iota
            al = alv[p][sl]
            be = bev[p][sl]

            @plsc.parallel_loop(0, CP, unroll=4)
            def _blend(w, r=r, al=al, be=be):
                wl = w + iota
                wl = jnp.where(wl >= CP, wl - CP, wl)
                tl = plsc.bitcast(plsc.load_gather(rtl, [r, wl]),
                                  jnp.bfloat16)
                tr = plsc.bitcast(plsc.load_gather(rtr, [r, wl]),
                                  jnp.bfloat16)
                bl = plsc.bitcast(plsc.load_gather(rbl, [r, wl]),
                                  jnp.bfloat16)
                br = plsc.bitcast(plsc.load_gather(rbr, [r, wl]),
                                  jnp.bfloat16)
                tl_e, tl_o = plsc.unpack(tl, format=fmt)
                tr_e, tr_o = plsc.unpack(tr, format=fmt)
                bl_e, bl_o = plsc.unpack(bl, format=fmt)
                br_e, br_o = plsc.unpack(br, format=fmt)
                top_e = tl_e + al * (tr_e - tl_e)
                bot_e = bl_e + al * (br_e - bl_e)
                plsc.store_scatter(ov, [2 * wl, r], top_e + be * (bot_e - top_e))
                top_o = tl_o + al * (tr_o - tl_o)
                bot_o = bl_o + al * (br_o - bl_o)
                plsc.store_scatter(ov, [2 * wl + 1, r],
                                   top_o + be * (bot_o - top_o))

        pltpu.async_copy(ov, out_hbm.at[oidx[p]], osem[p])

    y0 = r0 + jnp.int32(0)
    s0 = jnp.int32(0)
    flow_fire(y0, s0, 0)
    y1, s1 = _inc(y0, s0)
    flow_fire(y1, s1, 1)
    idx_and_fire(y0, s0, 0)

    def body(t, carry):
        ya, sa = carry
        yb, sb = _inc(ya, sa)
        yc, sc = _inc(yb, sb)
        yd, sd = _inc(yc, sc)
        last = t >= NCHUNK // 2 - 1
        ycc = jnp.where(last, ya, yc)
        scc = jnp.where(last, sa, sc)
        ydc = jnp.where(last, yb, yd)
        sdc = jnp.where(last, sb, sd)
        flow_fire(ycc, scc, 0)
        idx_and_fire(yb, sb, 1)
        blend_and_out(ya, sa, 0, t)
        flow_fire(ydc, sdc, 1)
        idx_and_fire(ycc, scc, 0)
        blend_and_out(yb, sb, 1, t)
        return yc, sc

    lax.fori_loop(0, NCHUNK // 2, body, (y0, s0))
    flow_wait(1)
    gather_wait(0)
    out_wait(0)
    out_wait(1)


_warp = pl.kernel(
    _warp_body,
    out_type=jax.ShapeDtypeStruct((B * C * H * SUBS, CHUNK), jnp.float32),
    compiler_params=pltpu.CompilerParams(
        needs_layout_passes=False, use_tc_tiling_on_sc=False),
    mesh=plsc.VectorSubcoreMesh(core_axis_name="c", subcore_axis_name="s"),
    scratch_types=[
        [pltpu.VMEM((CHUNK,), jnp.float32) for _ in range(2)],
        [pltpu.VMEM((CHUNK,), jnp.float32) for _ in range(2)],
        [pltpu.VMEM((CHUNK,), jnp.float32) for _ in range(2)],
        [pltpu.VMEM((CHUNK,), jnp.float32) for _ in range(2)],
        [[pltpu.VMEM((CHUNK,), jnp.int32) for _ in range(4)]
         for _ in range(2)],
        [pltpu.VMEM((C,), jnp.int32) for _ in range(2)],
        [[pltpu.VMEM((CHUNK, CW), jnp.int32) for _ in range(4)]
         for _ in range(2)],
        [pltpu.VMEM((C, CHUNK), jnp.float32) for _ in range(2)],
        [pltpu.SemaphoreType.DMA for _ in range(2)],
        [pltpu.SemaphoreType.DMA for _ in range(2)],
        [pltpu.SemaphoreType.DMA for _ in range(2)],
    ],
)


def kernel(input1, input2):
    t = input1.transpose(0, 2, 3, 1).astype(jnp.bfloat16)
    table = lax.bitcast_convert_type(t.reshape(V, CW, 2), jnp.int32)
    fx = input2[:, 0, :, :].reshape(V)
    fy = input2[:, 1, :, :].reshape(V)
    out = _warp(table, fx, fy)
    return out.reshape(B, C, H, W)


# speedup vs baseline: 2.2237x; 1.7630x over previous
import jax
import jax.numpy as jnp
from jax import lax
from jax.experimental import pallas as pl
from jax.experimental.pallas import tpu as pltpu
from jax.experimental.pallas import tpu_sc as plsc

B, C, H, W = 4, 96, 384, 384
HW = H * W
V = B * HW
L = 16
NC, NS = 2, 16
NW = NC * NS
RPW = H // (NW // B)
CHUNK = 128
SUBS = W // CHUNK
NCHUNK = RPW * SUBS
NG = CHUNK // L


def _inc(y, s):
    last = s == SUBS - 1
    return jnp.where(last, y + 1, y), jnp.where(last, 0, s + 1)


def _warp_body(table, fx, fy, out_hbm,
               fxv, fyv, alv, bev, idx, oidx, rows, outv,
               gsem, fsem, osem):
    wid = lax.axis_index("s") * NC + lax.axis_index("c")
    b = lax.shift_right_logical(wid, 3)
    r0 = (wid & 7) * RPW
    bhw = b * HW
    iota = lax.iota(jnp.int32, L)

    def flow_fire(y, s, p):
        off = bhw + y * W + s * CHUNK
        pltpu.async_copy(fx.at[pl.ds(off, CHUNK)], fxv[p], fsem[p])
        pltpu.async_copy(fy.at[pl.ds(off, CHUNK)], fyv[p], fsem[p])

    def flow_wait(p):
        pltpu.make_async_copy(fx.at[pl.ds(0, CHUNK)], fxv[p], fsem[p]).wait()
        pltpu.make_async_copy(fy.at[pl.ds(0, CHUNK)], fyv[p], fsem[p]).wait()

    def idx_and_fire(y, s, p):
        flow_wait(p)
        xoff = s * CHUNK
        yv = jnp.full((L,), y, jnp.int32)
        for k in range(NG):
            sl = pl.ds(k * L, L)
            xi = xoff + (k * L) + iota
            xf = xi.astype(jnp.float32) + fxv[p][sl]
            yf = yv.astype(jnp.float32) + fyv[p][sl]
            ix0 = xf.astype(jnp.int32)
            ix0 = jnp.where(ix0.astype(jnp.float32) > xf, ix0 - 1, ix0)
            iy0 = yf.astype(jnp.int32)
            iy0 = jnp.where(iy0.astype(jnp.float32) > yf, iy0 - 1, iy0)
            ixL = jnp.clip(ix0, 0, W - 1)
            iyT = jnp.clip(iy0, 0, H - 1)
            ixR = jnp.minimum(ixL + 1, W - 1)
            iyB = jnp.minimum(iyT + 1, H - 1)
            alv[p][sl] = xf - ixL.astype(jnp.float32)
            bev[p][sl] = yf - iyT.astype(jnp.float32)
            rowT = bhw + iyT * W
            rowB = bhw + iyB * W
            idx[p][0][sl] = rowT + ixL
            idx[p][1][sl] = rowT + ixR
            idx[p][2][sl] = rowB + ixL
            idx[p][3][sl] = rowB + ixR
        for q in range(4):
            pltpu.async_copy(table.at[idx[p][q]], rows[p][q], gsem[p])

    def gather_wait(p):
        for q in range(4):
            pltpu.make_async_copy(table.at[idx[p][q]], rows[p][q],
                                  gsem[p]).wait()

    def out_wait(p):
        pltpu.make_async_copy(outv[p], out_hbm.at[oidx[p]], osem[p]).wait()

    def blend_and_out(y, s, p, t):
        gather_wait(p)

        @pl.when(t > 0)
        def _():
            out_wait(p)

        yv = jnp.full((L,), y, jnp.int32)
        for g in range(C // L):
            cvec = g * L + iota
            oidx[p][pl.ds(g * L, L)] = ((b * C + cvec) * H + yv) * SUBS + s

        rtl, rtr, rbl, rbr = rows[p]
        ov = outv[p]
        for k in range(NG):
            sl = pl.ds(k * L, L)
            r = k * L + iota
            al = alv[p][sl]
            be = bev[p][sl]

            @plsc.parallel_loop(0, C, unroll=4)
            def _blend(c, r=r, al=al, be=be):
                cl = c + iota
                cl = jnp.where(cl >= C, cl - C, cl)
                tl = plsc.load_gather(rtl, [r, cl])
                tr = plsc.load_gather(rtr, [r, cl])
                bl = plsc.load_gather(rbl, [r, cl])
                br = plsc.load_gather(rbr, [r, cl])
                top = tl + al * (tr - tl)
                bot = bl + al * (br - bl)
                plsc.store_scatter(ov, [cl, r], top + be * (bot - top))

        pltpu.async_copy(ov, out_hbm.at[oidx[p]], osem[p])

    y0 = r0 + jnp.int32(0)
    s0 = jnp.int32(0)
    flow_fire(y0, s0, 0)
    y1, s1 = _inc(y0, s0)
    flow_fire(y1, s1, 1)
    idx_and_fire(y0, s0, 0)

    def body(t, carry):
        ya, sa = carry
        yb, sb = _inc(ya, sa)
        yc, sc = _inc(yb, sb)
        yd, sd = _inc(yc, sc)
        last = t >= NCHUNK // 2 - 1
        ycc = jnp.where(last, ya, yc)
        scc = jnp.where(last, sa, sc)
        ydc = jnp.where(last, yb, yd)
        sdc = jnp.where(last, sb, sd)
        flow_fire(ycc, scc, 0)
        idx_and_fire(yb, sb, 1)
        blend_and_out(ya, sa, 0, t)
        flow_fire(ydc, sdc, 1)
        idx_and_fire(ycc, scc, 0)
        blend_and_out(yb, sb, 1, t)
        return yc, sc

    lax.fori_loop(0, NCHUNK // 2, body, (y0, s0))
    flow_wait(1)
    gather_wait(0)
    out_wait(0)
    out_wait(1)


_warp = pl.kernel(
    _warp_body,
    out_type=jax.ShapeDtypeStruct((B * C * H * SUBS, CHUNK), jnp.float32),
    compiler_params=pltpu.CompilerParams(
        needs_layout_passes=False, use_tc_tiling_on_sc=False),
    mesh=plsc.VectorSubcoreMesh(core_axis_name="c", subcore_axis_name="s"),
    scratch_types=[
        [pltpu.VMEM((CHUNK,), jnp.float32) for _ in range(2)],
        [pltpu.VMEM((CHUNK,), jnp.float32) for _ in range(2)],
        [pltpu.VMEM((CHUNK,), jnp.float32) for _ in range(2)],
        [pltpu.VMEM((CHUNK,), jnp.float32) for _ in range(2)],
        [[pltpu.VMEM((CHUNK,), jnp.int32) for _ in range(4)]
         for _ in range(2)],
        [pltpu.VMEM((C,), jnp.int32) for _ in range(2)],
        [[pltpu.VMEM((CHUNK, C), jnp.float32) for _ in range(4)]
         for _ in range(2)],
        [pltpu.VMEM((C, CHUNK), jnp.float32) for _ in range(2)],
        [pltpu.SemaphoreType.DMA for _ in range(2)],
        [pltpu.SemaphoreType.DMA for _ in range(2)],
        [pltpu.SemaphoreType.DMA for _ in range(2)],
    ],
)


def kernel(input1, input2):
    table = input1.transpose(0, 2, 3, 1).reshape(V, C)
    fx = input2[:, 0, :, :].reshape(V)
    fy = input2[:, 1, :, :].reshape(V)
    out = _warp(table, fx, fy)
    return out.reshape(B, C, H, W)
